# baseline (speedup 1.0000x reference)
import functools

import jax
import jax.numpy as jnp
import numpy as np
from jax import lax
from jax.experimental import pallas as pl
from jax.experimental.pallas import tpu as pltpu
from jax.experimental.pallas import tpu_sc as plsc

_B, _NB, _D, _N = 8, 4, 256, 1024
_CD, _BINS = 512, 1024
_BPAIR = 2
_BTOK = _B * _N

_HI = jax.lax.Precision.HIGHEST


def _vq_band_body(x_ref, pit_ref, cb_ref, pot_ref,
                  cbpt_ref, codes_ref, losssum_ref,
                  cbsq_ref):
    bb = pl.program_id(0)

    @pl.when(bb == 0)
    def _():
        cb = cb_ref[0]
        pot = pot_ref[0]
        cbpt_ref[...] = jax.lax.dot_general(
            cb, pot, (((1,), (1,)), ((), ())), precision=_HI)
        cbsq_ref[...] = jnp.sum(cb * cb, axis=1, keepdims=True)
        losssum_ref[...] = jnp.zeros_like(losssum_ref)

    pit = pit_ref[0]
    iota = jax.lax.broadcasted_iota(jnp.int32, (_BINS, _N), 0)
    dn = (((1,), (0,)), ((), ()))
    loss = None
    for j in range(_BPAIR):
        x = x_ref[j, 0]
        z_t = jax.lax.dot_general(pit, x, dn)
        z_sq = jnp.sum(z_t * z_t, axis=0, keepdims=True)
        dots_t = jax.lax.dot_general(cb_ref[0], z_t, dn)
        dist_t = z_sq - 2.0 * dots_t + cbsq_ref[...]

        mind = jnp.min(dist_t, axis=0, keepdims=True)
        codes = jnp.min(jnp.where(dist_t == mind, iota, _BINS),
                        axis=0, keepdims=True)

        codes_ref[j] = codes
        s = jnp.sum(mind)
        loss = s if loss is None else loss + s

    losssum_ref[...] += loss


def _vq_band_call(q, x, pit, cb, pot):
    grid = (_B // _BPAIR,)
    return pl.pallas_call(
        _vq_band_body,
        grid=grid,
        in_specs=[
            pl.BlockSpec((_BPAIR, 1, _D, _N), lambda bb: (bb, q, 0, 0)),
            pl.BlockSpec((1, _CD, _D), lambda bb: (q, 0, 0)),
            pl.BlockSpec((1, _BINS, _CD), lambda bb: (q, 0, 0)),
            pl.BlockSpec((1, _D, _CD), lambda bb: (q, 0, 0)),
        ],
        out_specs=[
            pl.BlockSpec((_BINS, _D), lambda bb: (0, 0)),
            pl.BlockSpec((_BPAIR, 1, _N), lambda bb: (bb, 0, 0)),
            pl.BlockSpec((1, 1), lambda bb: (0, 0)),
        ],
        out_shape=[
            jax.ShapeDtypeStruct((_BINS, _D), jnp.float32),
            jax.ShapeDtypeStruct((_B, 1, _N), jnp.int32),
            jax.ShapeDtypeStruct((1, 1), jnp.float32),
        ],
        scratch_shapes=[
            pltpu.VMEM((_BINS, 1), jnp.float32),
        ],
        compiler_params=pltpu.CompilerParams(
            dimension_semantics=("arbitrary",),
        ),
    )(x, pit, cb, pot)


def _make_sc_gather():
    info = plsc.get_sparse_core_info()
    nw = info.num_cores * info.num_subcores
    b_per_w = _BTOK // nw
    ch = 128
    n_ch = b_per_w // ch
    mesh = plsc.VectorSubcoreMesh(core_axis_name="c", subcore_axis_name="s")

    @functools.partial(
        pl.kernel, mesh=mesh,
        out_type=jax.ShapeDtypeStruct((_BTOK, _D), jnp.float32),
        scratch_types=[
            pltpu.VMEM((b_per_w,), jnp.int32),
            pltpu.VMEM((ch, _D), jnp.float32),
            pltpu.VMEM((ch, _D), jnp.float32),
            pltpu.SemaphoreType.DMA,
            pltpu.SemaphoreType.DMA,
        ],
    )
    def sc_gather(table_hbm, idx_hbm, out_hbm, idx_v, rows_a, rows_b, sem_a,
                  sem_b):
        wid = lax.axis_index("s") * info.num_cores + lax.axis_index("c")
        base = wid * b_per_w
        pltpu.sync_copy(idx_hbm.at[pl.ds(base, b_per_w)], idx_v)
        bufs = ((rows_a, sem_a), (rows_b, sem_b))
        copies = [None, None]
        for k in range(n_ch + 1):
            if k < n_ch:
                rows, sem = bufs[k % 2]
                copies[k % 2] = pltpu.async_copy(
                    table_hbm.at[idx_v.at[pl.ds(k * ch, ch)]], rows, sem)
            if k > 0:
                rows, _ = bufs[(k - 1) % 2]
                copies[(k - 1) % 2].wait()
                pltpu.sync_copy(
                    rows, out_hbm.at[pl.ds(base + (k - 1) * ch, ch)])

    return sc_gather


_sc_gather = _make_sc_gather()


@jax.jit
def _vq_full(x, pit, cb, pot):
    quants, codes, loss = [], [], None
    for q in range(_NB):
        cbpt_q, codes_q, losssum_q = _vq_band_call(q, x, pit, cb, pot)
        rows_q = _sc_gather(cbpt_q, codes_q.reshape(_BTOK))
        quants.append(jnp.swapaxes(rows_q.reshape(_B, _N, _D), 1, 2))
        codes.append(codes_q)
        loss = losssum_q if loss is None else loss + losssum_q
    quant = jnp.stack(quants, axis=1)
    codes_out = jnp.concatenate(codes, axis=1)
    return quant, codes_out, loss


def kernel(x, sample_rate, proj_in, proj_out, codebook):
    pit = jnp.transpose(proj_in, (0, 2, 1))
    pot = jnp.transpose(proj_out, (0, 2, 1))
    quant, codes_out, losssum = _vq_full(x, pit, codebook, pot)
    loss = losssum[0, 0] / np.float32(_NB * _B * _N * _CD)
    bw = jnp.asarray(_NB * (np.log2(_BINS) * sample_rate / 1000.0),
                     dtype=x.dtype)
    return quant, codes_out, bw, loss


# speedup vs baseline: 2.2252x; 2.2252x over previous
import functools

import jax
import jax.numpy as jnp
import numpy as np
from jax.experimental import pallas as pl
from jax.experimental.pallas import tpu as pltpu

_B, _NB, _D, _N = 8, 4, 256, 1024
_CD, _BINS = 512, 1024
_BPAIR = 4

_HI = jax.lax.Precision.HIGHEST


def _vq_body(x_ref, pit_ref, cb_ref, pot_ref,
             quant_ref, codes_ref, losssum_ref,
             cbph_ref, cbpl_ref, cbsq_ref):
    q = pl.program_id(0)
    bb = pl.program_id(1)

    @pl.when(bb == 0)
    def _():
        cb = cb_ref[0]
        pot = pot_ref[0]
        cbp = jax.lax.dot_general(
            pot, cb, (((1,), (1,)), ((), ())), precision=_HI)
        hi = cbp.astype(jnp.bfloat16).astype(jnp.float32)
        cbph_ref[...] = hi
        cbpl_ref[...] = cbp - hi
        cbsq_ref[...] = jnp.sum(cb * cb, axis=1, keepdims=True)

    @pl.when(jnp.logical_and(q == 0, bb == 0))
    def _():
        losssum_ref[...] = jnp.zeros_like(losssum_ref)

    pit = pit_ref[0]
    iota = jax.lax.broadcasted_iota(jnp.int32, (_BINS, _N), 0)
    dn = (((1,), (0,)), ((), ()))
    loss = None
    for j in range(_BPAIR):
        x = x_ref[j, 0]
        z_t = jax.lax.dot_general(pit, x, dn)
        z_sq = jnp.sum(z_t * z_t, axis=0, keepdims=True)
        dots_t = jax.lax.dot_general(cb_ref[0], z_t, dn)
        dist_t = z_sq - 2.0 * dots_t + cbsq_ref[...]

        mind = jnp.min(dist_t, axis=0, keepdims=True)
        codes = jnp.min(jnp.where(dist_t == mind, iota, _BINS),
                        axis=0, keepdims=True)

        onehot = (iota == codes).astype(jnp.float32)
        quant_ref[j, 0] = (
            jax.lax.dot_general(cbph_ref[...], onehot, dn)
            + jax.lax.dot_general(cbpl_ref[...], onehot, dn))
        codes_ref[j, 0] = codes
        s = jnp.sum(mind)
        loss = s if loss is None else loss + s

    losssum_ref[...] += loss


@jax.jit
def _vq_call(x, pit, cb, pot):
    grid = (_NB, _B // _BPAIR)
    quant, codes4, losssum = pl.pallas_call(
        _vq_body,
        grid=grid,
        in_specs=[
            pl.BlockSpec((_BPAIR, 1, _D, _N), lambda q, bb: (bb, q, 0, 0)),
            pl.BlockSpec((1, _CD, _D), lambda q, bb: (q, 0, 0)),
            pl.BlockSpec((1, _BINS, _CD), lambda q, bb: (q, 0, 0)),
            pl.BlockSpec((1, _D, _CD), lambda q, bb: (q, 0, 0)),
        ],
        out_specs=[
            pl.BlockSpec((_BPAIR, 1, _D, _N), lambda q, bb: (bb, q, 0, 0)),
            pl.BlockSpec((_BPAIR, 1, 1, _N), lambda q, bb: (bb, q, 0, 0)),
            pl.BlockSpec((1, 1), lambda q, bb: (0, 0)),
        ],
        out_shape=[
            jax.ShapeDtypeStruct((_B, _NB, _D, _N), jnp.float32),
            jax.ShapeDtypeStruct((_B, _NB, 1, _N), jnp.int32),
            jax.ShapeDtypeStruct((1, 1), jnp.float32),
        ],
        scratch_shapes=[
            pltpu.VMEM((_D, _BINS), jnp.float32),
            pltpu.VMEM((_D, _BINS), jnp.float32),
            pltpu.VMEM((_BINS, 1), jnp.float32),
        ],
        compiler_params=pltpu.CompilerParams(
            dimension_semantics=("arbitrary", "arbitrary"),
        ),
    )(x, pit, cb, pot)
    return quant, codes4, losssum


def kernel(x, sample_rate, proj_in, proj_out, codebook):
    pit = jnp.transpose(proj_in, (0, 2, 1))
    pot = jnp.transpose(proj_out, (0, 2, 1))
    quant, codes4, losssum = _vq_call(x, pit, codebook, pot)
    codes_out = codes4.reshape(_B, _NB, _N)
    loss = losssum[0, 0] / np.float32(_NB * _B * _N * _CD)
    bw = jnp.asarray(_NB * (np.log2(_BINS) * sample_rate / 1000.0),
                     dtype=x.dtype)
    return quant, codes_out, bw, loss


# speedup vs baseline: 2.2872x; 1.0279x over previous
import functools

import jax
import jax.numpy as jnp
import numpy as np
from jax.experimental import pallas as pl
from jax.experimental.pallas import tpu as pltpu

_B, _NB, _D, _N = 8, 4, 256, 1024
_CD, _BINS = 512, 1024
_BPAIR = 4

_HI = jax.lax.Precision.HIGHEST


def _vq_body(x_ref, pit_ref, cb_ref, pot_ref,
             quant_ref, codes_ref, losssum_ref,
             cbph_ref, cbpl_ref, cbsq_ref):
    q = pl.program_id(0)
    bb = pl.program_id(1)

    @pl.when(bb == 0)
    def _():
        cb = cb_ref[0]
        pot = pot_ref[0]
        cbp = jax.lax.dot_general(
            pot, cb, (((1,), (1,)), ((), ())), precision=_HI)
        hi = cbp.astype(jnp.bfloat16).astype(jnp.float32)
        cbph_ref[...] = hi
        cbpl_ref[...] = cbp - hi
        cbsq_ref[...] = jnp.sum(cb * cb, axis=1, keepdims=True)

    @pl.when(jnp.logical_and(q == 0, bb == 0))
    def _():
        losssum_ref[...] = jnp.zeros_like(losssum_ref)

    pit = pit_ref[0]
    iotaf = jax.lax.broadcasted_iota(
        jnp.int32, (_BINS, _N), 0).astype(jnp.float32)
    dn = (((1,), (0,)), ((), ()))
    loss = None
    for j in range(_BPAIR):
        x = x_ref[j, 0]
        z_t = jax.lax.dot_general(pit, x, dn)
        z_sq = jnp.sum(z_t * z_t, axis=0, keepdims=True)
        dots_t = jax.lax.dot_general(cb_ref[0], z_t, dn)
        dist_t = z_sq - 2.0 * dots_t + cbsq_ref[...]

        mind = jnp.min(dist_t, axis=0, keepdims=True)
        codesf = jnp.min(jnp.where(dist_t == mind, iotaf, float(_BINS)),
                         axis=0, keepdims=True)

        onehot = (iotaf == codesf).astype(jnp.float32)
        quant_ref[j, 0] = (
            jax.lax.dot_general(cbph_ref[...], onehot, dn)
            + jax.lax.dot_general(cbpl_ref[...], onehot, dn))
        codes_ref[j, 0] = codesf.astype(jnp.int32)
        s = jnp.sum(mind)
        loss = s if loss is None else loss + s

    losssum_ref[...] += loss


@jax.jit
def _vq_call(x, pit, cb, pot):
    grid = (_NB, _B // _BPAIR)
    quant, codes4, losssum = pl.pallas_call(
        _vq_body,
        grid=grid,
        in_specs=[
            pl.BlockSpec((_BPAIR, 1, _D, _N), lambda q, bb: (bb, q, 0, 0)),
            pl.BlockSpec((1, _CD, _D), lambda q, bb: (q, 0, 0)),
            pl.BlockSpec((1, _BINS, _CD), lambda q, bb: (q, 0, 0)),
            pl.BlockSpec((1, _D, _CD), lambda q, bb: (q, 0, 0)),
        ],
        out_specs=[
            pl.BlockSpec((_BPAIR, 1, _D, _N), lambda q, bb: (bb, q, 0, 0)),
            pl.BlockSpec((_BPAIR, 1, 1, _N), lambda q, bb: (bb, q, 0, 0)),
            pl.BlockSpec((1, 1), lambda q, bb: (0, 0)),
        ],
        out_shape=[
            jax.ShapeDtypeStruct((_B, _NB, _D, _N), jnp.float32),
            jax.ShapeDtypeStruct((_B, _NB, 1, _N), jnp.int32),
            jax.ShapeDtypeStruct((1, 1), jnp.float32),
        ],
        scratch_shapes=[
            pltpu.VMEM((_D, _BINS), jnp.float32),
            pltpu.VMEM((_D, _BINS), jnp.float32),
            pltpu.VMEM((_BINS, 1), jnp.float32),
        ],
        compiler_params=pltpu.CompilerParams(
            dimension_semantics=("arbitrary", "arbitrary"),
        ),
    )(x, pit, cb, pot)
    return quant, codes4, losssum


def kernel(x, sample_rate, proj_in, proj_out, codebook):
    pit = jnp.transpose(proj_in, (0, 2, 1))
    pot = jnp.transpose(proj_out, (0, 2, 1))
    quant, codes4, losssum = _vq_call(x, pit, codebook, pot)
    codes_out = codes4.reshape(_B, _NB, _N)
    loss = losssum[0, 0] / np.float32(_NB * _B * _N * _CD)
    bw = jnp.asarray(_NB * (np.log2(_BINS) * sample_rate / 1000.0),
                     dtype=x.dtype)
    return quant, codes_out, bw, loss


# speedup vs baseline: 2.3256x; 1.0168x over previous
import functools

import jax
import jax.numpy as jnp
import numpy as np
from jax.experimental import pallas as pl
from jax.experimental.pallas import tpu as pltpu

_B, _NB, _D, _N = 8, 4, 256, 1024
_CD, _BINS = 512, 1024
_BPAIR = 4

_HI = jax.lax.Precision.HIGHEST


def _vq_body(x_ref, pit_ref, cb_ref, pot_ref,
             quant_ref, codes_ref, losssum_ref,
             cbphl_ref, cbsq_ref):
    q = pl.program_id(0)
    bb = pl.program_id(1)

    @pl.when(bb == 0)
    def _():
        cb = cb_ref[0]
        pot = pot_ref[0]
        cbp = jax.lax.dot_general(
            pot, cb, (((1,), (1,)), ((), ())), precision=_HI)
        hi = cbp.astype(jnp.bfloat16).astype(jnp.float32)
        cbphl_ref[0:_D] = hi
        cbphl_ref[_D:2 * _D] = cbp - hi
        cbsq_ref[...] = jnp.sum(cb * cb, axis=1, keepdims=True)

    @pl.when(jnp.logical_and(q == 0, bb == 0))
    def _():
        losssum_ref[...] = jnp.zeros_like(losssum_ref)

    pit = pit_ref[0]
    iotaf = jax.lax.broadcasted_iota(
        jnp.int32, (_BINS, _N), 0).astype(jnp.float32)
    dn = (((1,), (0,)), ((), ()))
    loss = None
    for j in range(_BPAIR):
        x = x_ref[j, 0]
        z_t = jax.lax.dot_general(pit, x, dn)
        z_sq = jnp.sum(z_t * z_t, axis=0, keepdims=True)
        dots_t = jax.lax.dot_general(cb_ref[0], z_t, dn)
        dist_t = z_sq - 2.0 * dots_t + cbsq_ref[...]

        mind = jnp.min(dist_t, axis=0, keepdims=True)
        codesf = jnp.min(jnp.where(dist_t == mind, iotaf, float(_BINS)),
                         axis=0, keepdims=True)

        onehot = (iotaf == codesf).astype(jnp.float32)
        r = jax.lax.dot_general(cbphl_ref[...], onehot, dn)
        quant_ref[j, 0] = r[0:_D] + r[_D:2 * _D]
        codes_ref[j, 0] = codesf.astype(jnp.int32)
        s = jnp.sum(mind)
        loss = s if loss is None else loss + s

    losssum_ref[...] += loss


@jax.jit
def _vq_call(x, pit, cb, pot):
    grid = (_NB, _B // _BPAIR)
    quant, codes4, losssum = pl.pallas_call(
        _vq_body,
        grid=grid,
        in_specs=[
            pl.BlockSpec((_BPAIR, 1, _D, _N), lambda q, bb: (bb, q, 0, 0)),
            pl.BlockSpec((1, _CD, _D), lambda q, bb: (q, 0, 0)),
            pl.BlockSpec((1, _BINS, _CD), lambda q, bb: (q, 0, 0)),
            pl.BlockSpec((1, _D, _CD), lambda q, bb: (q, 0, 0)),
        ],
        out_specs=[
            pl.BlockSpec((_BPAIR, 1, _D, _N), lambda q, bb: (bb, q, 0, 0)),
            pl.BlockSpec((_BPAIR, 1, 1, _N), lambda q, bb: (bb, q, 0, 0)),
            pl.BlockSpec((1, 1), lambda q, bb: (0, 0)),
        ],
        out_shape=[
            jax.ShapeDtypeStruct((_B, _NB, _D, _N), jnp.float32),
            jax.ShapeDtypeStruct((_B, _NB, 1, _N), jnp.int32),
            jax.ShapeDtypeStruct((1, 1), jnp.float32),
        ],
        scratch_shapes=[
            pltpu.VMEM((2 * _D, _BINS), jnp.float32),
            pltpu.VMEM((_BINS, 1), jnp.float32),
        ],
        compiler_params=pltpu.CompilerParams(
            dimension_semantics=("arbitrary", "arbitrary"),
        ),
    )(x, pit, cb, pot)
    return quant, codes4, losssum


def kernel(x, sample_rate, proj_in, proj_out, codebook):
    pit = jnp.transpose(proj_in, (0, 2, 1))
    pot = jnp.transpose(proj_out, (0, 2, 1))
    quant, codes4, losssum = _vq_call(x, pit, codebook, pot)
    codes_out = codes4.reshape(_B, _NB, _N)
    loss = losssum[0, 0] / np.float32(_NB * _B * _N * _CD)
    bw = jnp.asarray(_NB * (np.log2(_BINS) * sample_rate / 1000.0),
                     dtype=x.dtype)
    return quant, codes_out, bw, loss
